# Initial kernel scaffold; baseline (speedup 1.0000x reference)
import jax
import jax.numpy as jnp
from jax.experimental import pallas as pl


def kernel(nodes, edge_index, edge_weights, neighbor_idx, emb_table, W1, b1, W2, b2):
    raise NotImplementedError("write your pallas kernel here")


# speedup vs baseline: 10.8896x; 10.8896x over previous
import functools

import jax
import jax.numpy as jnp
from jax import lax
from jax.experimental import pallas as pl
from jax.experimental.pallas import tpu as pltpu
from jax.experimental.pallas import tpu_sc as plsc

N = 10000
E = 320000
D = 128
K = 4
H = 128

NC = 2
NS = 16
NW = NC * NS
L = 16

EC = 128
EPT = -(-(E // NW) // EC)
EPAD = NW * EPT * EC
GPT = -(-(N * K // NW) // EC)
GPAD = NW * GPT * EC

NPAD = 10240
RPS = NPAD // NS

_mesh = plsc.VectorSubcoreMesh(core_axis_name="c", subcore_axis_name="s")
_sc_params = pltpu.CompilerParams(needs_layout_passes=False)


@functools.partial(
    pl.kernel,
    out_type=(
        jax.ShapeDtypeStruct((GPAD, D), jnp.float32),
        jax.ShapeDtypeStruct((NC, NPAD), jnp.float32),
    ),
    mesh=_mesh,
    scratch_types=[
        pltpu.VMEM((N,), jnp.int32),
        pltpu.VMEM((GPT, EC), jnp.int32),
        pltpu.VMEM((GPT, EC), jnp.int32),
        pltpu.VMEM((EC, D), jnp.float32),
        pltpu.VMEM((EPT, EC), jnp.int32),
        pltpu.VMEM((EPT, EC), jnp.float32),
        pltpu.VMEM_SHARED((NPAD,), jnp.float32),
        pltpu.SemaphoreType.DMA,
    ],
    compiler_params=_sc_params,
)
def _sc_gather_deg(nodes_hbm, nbr_hbm, dst_hbm, w_hbm, zeros1_hbm, emb_hbm,
                   comb_hbm, degp_hbm,
                   nodes_v, nb_v, idx_v, rows_v, dst_v, w_v, deg_sh, sem):
    c = lax.axis_index("c")
    s = lax.axis_index("s")
    wid = s * NC + c

    pltpu.sync_copy(zeros1_hbm.at[pl.ds(s * RPS, RPS)],
                    deg_sh.at[pl.ds(s * RPS, RPS)])

    pltpu.sync_copy(nodes_hbm, nodes_v)
    pltpu.sync_copy(nbr_hbm.at[wid], nb_v)
    pltpu.sync_copy(dst_hbm.at[wid], dst_v)
    pltpu.sync_copy(w_hbm.at[wid], w_v)

    def _xlate(j, _):
        def _grp(g, _):
            nb16 = nb_v[j, pl.ds(g * L, L)]
            idx_v[j, pl.ds(g * L, L)] = plsc.load_gather(nodes_v, [nb16])
            return 0
        lax.fori_loop(0, EC // L, _grp, 0)
        return 0
    lax.fori_loop(0, GPT, _xlate, 0)

    def _rows(j, _):
        pltpu.async_copy(emb_hbm.at[idx_v.at[j]], rows_v, sem).wait()
        pltpu.sync_copy(rows_v, comb_hbm.at[pl.ds((wid * GPT + j) * EC, EC)])
        return 0
    lax.fori_loop(0, GPT, _rows, 0)

    plsc.subcore_barrier()
    def _deg(j, _):
        pltpu.sync_copy(w_v.at[j], deg_sh.at[dst_v.at[j]], add=True)
        return 0
    lax.fori_loop(0, EPT, _deg, 0)
    plsc.subcore_barrier()
    pltpu.sync_copy(deg_sh.at[pl.ds(s * RPS, RPS)],
                    degp_hbm.at[c, pl.ds(s * RPS, RPS)])


@functools.partial(
    pl.kernel,
    out_type=jax.ShapeDtypeStruct((NC, NPAD, D), jnp.float32),
    mesh=_mesh,
    scratch_types=[
        pltpu.VMEM((EPT, EC), jnp.int32),
        pltpu.VMEM((EPT, EC), jnp.int32),
        pltpu.VMEM((EPT, EC), jnp.float32),
        pltpu.VMEM((EC, D), jnp.float32),
        pltpu.VMEM_SHARED((NPAD, D), jnp.float32),
        pltpu.SemaphoreType.DMA,
    ],
    compiler_params=_sc_params,
)
def _sc_mp(hs_hbm, src_hbm, dst_hbm, w_hbm, zeros2_hbm, part_hbm,
           src_v, dst_v, w_v, rows_v, agg_sh, sem):
    c = lax.axis_index("c")
    s = lax.axis_index("s")
    wid = s * NC + c

    pltpu.sync_copy(zeros2_hbm.at[pl.ds(s * RPS, RPS)],
                    agg_sh.at[pl.ds(s * RPS, RPS)])
    pltpu.sync_copy(src_hbm.at[wid], src_v)
    pltpu.sync_copy(dst_hbm.at[wid], dst_v)
    pltpu.sync_copy(w_hbm.at[wid], w_v)
    plsc.subcore_barrier()

    def _chunk(j, _):
        pltpu.async_copy(hs_hbm.at[src_v.at[j]], rows_v, sem).wait()

        def _grp(g, _):
            w16 = w_v[j, pl.ds(g * L, L)]
            for i in range(L):
                spl = lax.broadcast(w16[i], (L,))
                e = g * L + i
                for cb in range(D // L):
                    rows_v[e, pl.ds(cb * L, L)] = (
                        rows_v[e, pl.ds(cb * L, L)] * spl)
            return 0
        lax.fori_loop(0, EC // L, _grp, 0)

        pltpu.sync_copy(rows_v, agg_sh.at[dst_v.at[j]], add=True)
        return 0
    lax.fori_loop(0, EPT, _chunk, 0)

    plsc.subcore_barrier()
    pltpu.sync_copy(agg_sh.at[pl.ds(s * RPS, RPS)],
                    part_hbm.at[c, pl.ds(s * RPS, RPS)])


_R = 1024


def _tc1_body(comb_ref, degp_ref, w1_ref, out_ref):
    deg = degp_ref[0, :] + degp_ref[1, :] + 1.0
    dinv = lax.rsqrt(deg)
    h = jnp.dot(comb_ref[...], w1_ref[...], preferred_element_type=jnp.float32)
    out_ref[...] = h * dinv[:, None]


def _tc1(comb, degp, w1):
    return pl.pallas_call(
        _tc1_body,
        grid=(NPAD // _R,),
        in_specs=[
            pl.BlockSpec((_R, K * D), lambda i: (i, 0)),
            pl.BlockSpec((NC, _R), lambda i: (0, i)),
            pl.BlockSpec((K * D, H), lambda i: (0, 0)),
        ],
        out_specs=pl.BlockSpec((_R, H), lambda i: (i, 0)),
        out_shape=jax.ShapeDtypeStruct((NPAD, H), jnp.float32),
    )(comb, degp, w1)


def _tc2_body(part_ref, hs_ref, degp_ref, b1_ref, w2_ref, out_ref):
    deg = degp_ref[0, :] + degp_ref[1, :] + 1.0
    dinv = lax.rsqrt(deg)
    p = part_ref[0] + part_ref[1] + hs_ref[...]
    x2 = jnp.maximum(p * dinv[:, None] + b1_ref[...], 0.0)
    h = jnp.dot(x2, w2_ref[...], preferred_element_type=jnp.float32)
    out_ref[...] = h * dinv[:, None]


def _tc2(part, hs, degp, b1, w2):
    return pl.pallas_call(
        _tc2_body,
        grid=(NPAD // _R,),
        in_specs=[
            pl.BlockSpec((NC, _R, H), lambda i: (0, i, 0)),
            pl.BlockSpec((_R, H), lambda i: (i, 0)),
            pl.BlockSpec((NC, _R), lambda i: (0, i)),
            pl.BlockSpec((1, H), lambda i: (0, 0)),
            pl.BlockSpec((H, D), lambda i: (0, 0)),
        ],
        out_specs=pl.BlockSpec((_R, D), lambda i: (i, 0)),
        out_shape=jax.ShapeDtypeStruct((NPAD, D), jnp.float32),
    )(part, hs, degp, b1, w2)


def _tc3_body(part_ref, hs_ref, degp_ref, b2_ref, out_ref):
    deg = degp_ref[0, :] + degp_ref[1, :] + 1.0
    dinv = lax.rsqrt(deg)
    p = part_ref[0] + part_ref[1] + hs_ref[...]
    out_ref[...] = p * dinv[:, None] + b2_ref[...]


def _tc3(part, hs, degp, b2):
    return pl.pallas_call(
        _tc3_body,
        grid=(NPAD // _R,),
        in_specs=[
            pl.BlockSpec((NC, _R, D), lambda i: (0, i, 0)),
            pl.BlockSpec((_R, D), lambda i: (i, 0)),
            pl.BlockSpec((NC, _R), lambda i: (0, i)),
            pl.BlockSpec((1, D), lambda i: (0, 0)),
        ],
        out_specs=pl.BlockSpec((_R, D), lambda i: (i, 0)),
        out_shape=jax.ShapeDtypeStruct((NPAD, D), jnp.float32),
    )(part, hs, degp, b2)


def kernel(nodes, edge_index, edge_weights, neighbor_idx, emb_table,
           W1, b1, W2, b2):
    i32 = jnp.int32
    src = jnp.concatenate(
        [edge_index[0], jnp.zeros((EPAD - E,), i32)]).reshape(NW, EPT, EC)
    dst = jnp.concatenate(
        [edge_index[1], jnp.zeros((EPAD - E,), i32)]).reshape(NW, EPT, EC)
    w = jnp.concatenate(
        [edge_weights, jnp.zeros((EPAD - E,), jnp.float32)]
    ).reshape(NW, EPT, EC)
    nbr = jnp.concatenate(
        [neighbor_idx.reshape(-1), jnp.zeros((GPAD - N * K,), i32)]
    ).reshape(NW, GPT, EC)
    zeros1 = jnp.zeros((NPAD,), jnp.float32)
    zeros2 = jnp.zeros((NPAD, D), jnp.float32)

    comb_rows, degp = _sc_gather_deg(nodes, nbr, dst, w, zeros1, emb_table)
    comb = comb_rows.reshape(NPAD, K * D)

    hs1 = _tc1(comb, degp, W1)
    part1 = _sc_mp(hs1, src, dst, w, zeros2)
    hs2 = _tc2(part1, hs1, degp, b1.reshape(1, H), W2)
    part2 = _sc_mp(hs2, src, dst, w, zeros2)
    out = _tc3(part2, hs2, degp, b2.reshape(1, D))
    return out[:N]
